# Initial kernel scaffold; baseline (speedup 1.0000x reference)
import jax
import jax.numpy as jnp
from jax.experimental import pallas as pl


def kernel(sent_id, words, heads_ids, emb, W1, b1, W2, b2, W3, b3, c1w, c1b, c2w, c2b):
    raise NotImplementedError("write your pallas kernel here")


# speedup vs baseline: 59.1599x; 59.1599x over previous
import functools

import jax
import jax.numpy as jnp
from jax import lax
from jax.experimental import pallas as pl
from jax.experimental.pallas import tpu as pltpu

MAX_DIST = 128
MAX_SENT_LEN = 256
NEG_INF = float('-inf')


def _wordproj_kernel(x_ref, w_ref, o_ref):
    o_ref[...] = jnp.dot(x_ref[...], w_ref[...],
                         preferred_element_type=jnp.float32)


TBL = 536
TXT = 544


def _trev_kernel(emb_ref, w1e_ref, o_ref):
    ep = jnp.dot(emb_ref[...], w1e_ref[...],
                 preferred_element_type=jnp.float32)
    j = lax.broadcasted_iota(jnp.int32, (TXT, MAX_DIST), 0)
    i = lax.broadcasted_iota(jnp.int32, (TXT, MAX_DIST), 1)
    e = (263 + (MAX_DIST - 2) // 2) - j
    tgt = jnp.where((e >= 0) & (e <= MAX_DIST - 2), e, MAX_DIST - 1)
    sel = (i == tgt).astype(jnp.float32)
    text = jnp.dot(sel, ep, preferred_element_type=jnp.float32)
    for v in range(8):
        o_ref[v, :, :] = text[v:v + TBL, :]


def _shift_dn(x):
    z = jnp.zeros((1, x.shape[1]), jnp.float32)
    return jnp.concatenate([z, x[:-1, :]], axis=0)


def _shift_up(x):
    z = jnp.zeros((1, x.shape[1]), jnp.float32)
    return jnp.concatenate([x[1:, :], z], axis=0)


def _head_kernel(starts_r, lens_r, offs_r, hids_r, ml_r,
                 wp_hbm, trev_ref, hw_ref, w1h_ref, w2_ref, w3_ref,
                 k10_ref, k11_ref, k12_ref, k20_ref, k21_ref, k22_ref,
                 b1_ref, b2_ref, b3_ref, c1b_ref, c2b_ref,
                 init_hbm, out_hbm,
                 wp_s, win_s, sem1, sem2):
    del init_hbm
    h = pl.program_id(0)
    start = starts_r[h]
    ln = lens_r[h]
    off = offs_r[h]
    ml = ml_r[0]

    L = MAX_SENT_LEN
    LP = L + 8
    a = (start // 8) * 8
    r = start - a
    cp = pltpu.make_async_copy(wp_hbm.at[pl.ds(a, LP), :], wp_s, sem1)
    cp.start()
    cp.wait()

    base = jnp.clip(263 - off - r, 0, TBL - LP)
    b_hi = (base // 8) * 8
    b_lo = base - b_hi
    e1 = trev_ref[b_lo, pl.ds(b_hi, LP), :]
    hp = jnp.dot(hw_ref[0], w1h_ref[...],
                 preferred_element_type=jnp.float32)
    pos = lax.broadcasted_iota(jnp.int32, (LP, 1), 0) - r

    b1 = b1_ref[...]
    pre = wp_s[...] + e1 + hp + b1
    h1 = jnp.where((pos >= 0) & (pos < ln),
                   jnp.maximum(pre, 0.0), jnp.maximum(b1, 0.0))
    h2 = jnp.maximum(
        jnp.dot(h1, w2_ref[...], preferred_element_type=jnp.float32)
        + b2_ref[...], 0.0)
    h3 = jnp.dot(h2, w3_ref[...], preferred_element_type=jnp.float32) \
        + b3_ref[...]
    h3 = jnp.where((pos >= 0) & (pos < ml), h3, 0.0)

    r1 = (jnp.dot(_shift_dn(h3), k10_ref[...],
                  preferred_element_type=jnp.float32)
          + jnp.dot(h3, k11_ref[...], preferred_element_type=jnp.float32)
          + jnp.dot(_shift_up(h3), k12_ref[...],
                    preferred_element_type=jnp.float32)
          + c1b_ref[...])
    r1 = jnp.where((pos >= 0) & (pos < ml), r1, 0.0)
    z = (jnp.dot(_shift_dn(r1), k20_ref[...],
                 preferred_element_type=jnp.float32)
         + jnp.dot(r1, k21_ref[...], preferred_element_type=jnp.float32)
         + jnp.dot(_shift_up(r1), k22_ref[...],
                   preferred_element_type=jnp.float32)
         + c2b_ref[...])

    s0 = z[:, 0:1]
    s1 = z[:, 1:2]
    v0 = jnp.where((pos >= 0) & (pos < ln) & (pos <= off), s0, NEG_INF)
    v1 = jnp.where((pos < ln) & (pos >= off), s1, NEG_INF)
    win_s[...] = jnp.concatenate([v0, v1], axis=1)

    cp2 = pltpu.make_async_copy(win_s, out_hbm.at[h, pl.ds(a, LP), :],
                                sem2)
    cp2.start()
    cp2.wait()


@functools.partial(jax.jit, static_argnames=())
def _impl(sent_id, words, heads_ids, emb, W1, b1, W2, b2, W3, b3,
          c1w, c1b, c2w, c2b):
    n_words = words.shape[0]
    n_heads = heads_ids.shape[0]
    input_size = words.shape[1]
    hidden = W1.shape[0]
    h2dim = W2.shape[0]
    demb = W3.shape[0]
    L = MAX_SENT_LEN

    heads_ids = heads_ids.astype(jnp.int32)
    sid_h = sent_id[heads_ids]
    starts = jnp.searchsorted(sent_id, sid_h, side='left').astype(jnp.int32)
    ends = jnp.searchsorted(sent_id, sid_h, side='right').astype(jnp.int32)
    lens = ends - starts
    offs = heads_ids - starts
    max_len = jnp.max(lens).reshape((1,))

    W1hT = W1[:, :input_size].T
    W1wT = W1[:, input_size:2 * input_size].T
    W1eT = W1[:, 2 * input_size:].T
    W2T = W2.T
    W3T = W3.T
    cc = c1w.shape[0]
    k1 = [jnp.pad(c1w[:, :, k].T, ((0, 0), (0, 128 - cc))) for k in range(3)]
    k2 = [jnp.pad(c2w[:, :, k].T, ((0, 128 - cc), (0, 126))) for k in range(3)]
    b1r = b1.reshape(1, hidden)
    b2r = b2.reshape(1, h2dim)
    b3r = b3.reshape(1, demb)
    c1br = jnp.pad(c1b, (0, 128 - cc)).reshape(1, 128)
    c2br = jnp.pad(c2b, (0, 126)).reshape(1, 128)

    npad = n_words + L
    nblk = n_words // L
    wordproj = pl.pallas_call(
        _wordproj_kernel,
        grid=(nblk + 1,),
        in_specs=[
            pl.BlockSpec((L, input_size),
                         lambda i: (jnp.minimum(i, nblk - 1), 0)),
            pl.BlockSpec((input_size, hidden), lambda i: (0, 0)),
        ],
        out_specs=pl.BlockSpec((L, hidden), lambda i: (i, 0)),
        out_shape=jax.ShapeDtypeStruct((npad, hidden), jnp.float32),
    )(words, W1wT)

    trev = pl.pallas_call(
        _trev_kernel,
        in_specs=[pl.BlockSpec(emb.shape, lambda: (0, 0)),
                  pl.BlockSpec((demb, hidden), lambda: (0, 0))],
        out_specs=pl.BlockSpec((8, TBL, hidden), lambda: (0, 0, 0)),
        out_shape=jax.ShapeDtypeStruct((8, TBL, hidden), jnp.float32),
    )(emb, W1eT)

    init = jnp.full((n_heads, npad, 2), NEG_INF, dtype=jnp.float32)

    grid_spec = pltpu.PrefetchScalarGridSpec(
        num_scalar_prefetch=5,
        grid=(n_heads,),
        in_specs=[
            pl.BlockSpec(memory_space=pl.ANY),
            pl.BlockSpec((8, TBL, hidden), lambda h, *_: (0, 0, 0)),
            pl.BlockSpec((1, 1, input_size),
                         lambda h, s, l, o, hid, ml: (hid[h], 0, 0)),
            pl.BlockSpec((input_size, hidden), lambda h, *_: (0, 0)),
            pl.BlockSpec((hidden, h2dim), lambda h, *_: (0, 0)),
            pl.BlockSpec((h2dim, demb), lambda h, *_: (0, 0)),
            pl.BlockSpec((demb, 128), lambda h, *_: (0, 0)),
            pl.BlockSpec((demb, 128), lambda h, *_: (0, 0)),
            pl.BlockSpec((demb, 128), lambda h, *_: (0, 0)),
            pl.BlockSpec((128, 128), lambda h, *_: (0, 0)),
            pl.BlockSpec((128, 128), lambda h, *_: (0, 0)),
            pl.BlockSpec((128, 128), lambda h, *_: (0, 0)),
            pl.BlockSpec((1, hidden), lambda h, *_: (0, 0)),
            pl.BlockSpec((1, h2dim), lambda h, *_: (0, 0)),
            pl.BlockSpec((1, demb), lambda h, *_: (0, 0)),
            pl.BlockSpec((1, 128), lambda h, *_: (0, 0)),
            pl.BlockSpec((1, 128), lambda h, *_: (0, 0)),
            pl.BlockSpec(memory_space=pl.ANY),
        ],
        out_specs=pl.BlockSpec(memory_space=pl.ANY),
        scratch_shapes=[
            pltpu.VMEM((L + 8, hidden), jnp.float32),
            pltpu.VMEM((L + 8, 2), jnp.float32),
            pltpu.SemaphoreType.DMA,
            pltpu.SemaphoreType.DMA,
        ],
    )

    out = pl.pallas_call(
        _head_kernel,
        grid_spec=grid_spec,
        out_shape=jax.ShapeDtypeStruct((n_heads, npad, 2), jnp.float32),
        input_output_aliases={22: 0},
        compiler_params=pltpu.CompilerParams(
            dimension_semantics=("arbitrary",)),
    )(starts, lens, offs, heads_ids, max_len,
      wordproj, trev, words.reshape(n_words, 1, input_size), W1hT, W2T, W3T,
      k1[0], k1[1], k1[2], k2[0], k2[1], k2[2],
      b1r, b2r, b3r, c1br, c2br, init)

    return out[:, :n_words, :]


def kernel(sent_id, words, heads_ids, emb, W1, b1, W2, b2, W3, b3,
           c1w, c1b, c2w, c2b):
    return _impl(sent_id, words, heads_ids, emb, W1, b1, W2, b2, W3, b3,
                 c1w, c1b, c2w, c2b)


# speedup vs baseline: 67.1315x; 1.1347x over previous
import functools

import jax
import jax.numpy as jnp
from jax import lax
from jax.experimental import pallas as pl
from jax.experimental.pallas import tpu as pltpu

MAX_DIST = 128
MAX_SENT_LEN = 256
NEG_INF = float('-inf')


def _wordproj_kernel(x_ref, w_ref, o_ref):
    o_ref[...] = jnp.dot(x_ref[...], w_ref[...],
                         preferred_element_type=jnp.float32)


TBL = 536
TXT = 544


def _trev_kernel(emb_ref, w1e_ref, o_ref):
    ep = jnp.dot(emb_ref[...], w1e_ref[...],
                 preferred_element_type=jnp.float32)
    j = lax.broadcasted_iota(jnp.int32, (TXT, MAX_DIST), 0)
    i = lax.broadcasted_iota(jnp.int32, (TXT, MAX_DIST), 1)
    e = (263 + (MAX_DIST - 2) // 2) - j
    tgt = jnp.where((e >= 0) & (e <= MAX_DIST - 2), e, MAX_DIST - 1)
    sel = (i == tgt).astype(jnp.float32)
    text = jnp.dot(sel, ep, preferred_element_type=jnp.float32)
    for v in range(8):
        o_ref[v, :, :] = text[v:v + TBL, :]


def _shift_dn(x):
    z = jnp.zeros((1, x.shape[1]), jnp.float32)
    return jnp.concatenate([z, x[:-1, :]], axis=0)


def _shift_up(x):
    z = jnp.zeros((1, x.shape[1]), jnp.float32)
    return jnp.concatenate([x[1:, :], z], axis=0)


def _head_kernel(starts_r, lens_r, offs_r, hids_r, ml_r,
                 wp_hbm, trev_ref, hw_ref, w1h_ref, w2_ref, w3_ref,
                 k10_ref, k11_ref, k12_ref, k20_ref, k21_ref, k22_ref,
                 b1_ref, b2_ref, b3_ref, c1b_ref, c2b_ref,
                 init_hbm, out_hbm,
                 wp_s, win_s, sem1, sem2):
    del init_hbm
    h = pl.program_id(0)
    n = pl.num_programs(0)
    start = starts_r[h]
    ln = lens_r[h]
    off = offs_r[h]
    ml = ml_r[0]

    L = MAX_SENT_LEN
    LP = L + 8
    a = (start // 8) * 8
    r = start - a
    slot = lax.rem(h, 2)
    nslot = 1 - slot

    def in_copy(idx, sl):
        aa = (starts_r[idx] // 8) * 8
        return pltpu.make_async_copy(wp_hbm.at[pl.ds(aa, LP), :],
                                     wp_s.at[sl], sem1.at[sl])

    def out_copy(base_row, sl):
        return pltpu.make_async_copy(win_s.at[sl],
                                     out_hbm.at[0, pl.ds(base_row, LP), :],
                                     sem2.at[sl])

    @pl.when(h == 0)
    def _():
        in_copy(h, slot).start()

    @pl.when(h + 1 < n)
    def _():
        in_copy(h + 1, nslot).start()

    in_copy(h, slot).wait()

    base = jnp.clip(263 - off - r, 0, TBL - LP)
    b_hi = (base // 8) * 8
    b_lo = base - b_hi
    e1 = trev_ref[b_lo, pl.ds(b_hi, LP), :]
    hp = jnp.dot(hw_ref[0], w1h_ref[...],
                 preferred_element_type=jnp.float32)
    pos = lax.broadcasted_iota(jnp.int32, (LP, 1), 0) - r

    b1 = b1_ref[...]
    pre = wp_s[slot] + e1 + hp + b1
    h1 = jnp.where((pos >= 0) & (pos < ln),
                   jnp.maximum(pre, 0.0), jnp.maximum(b1, 0.0))
    h2 = jnp.maximum(
        jnp.dot(h1, w2_ref[...], preferred_element_type=jnp.float32)
        + b2_ref[...], 0.0)
    h3 = jnp.dot(h2, w3_ref[...], preferred_element_type=jnp.float32) \
        + b3_ref[...]
    h3 = jnp.where((pos >= 0) & (pos < ml), h3, 0.0)

    r1 = (jnp.dot(_shift_dn(h3), k10_ref[...],
                  preferred_element_type=jnp.float32)
          + jnp.dot(h3, k11_ref[...], preferred_element_type=jnp.float32)
          + jnp.dot(_shift_up(h3), k12_ref[...],
                    preferred_element_type=jnp.float32)
          + c1b_ref[...])
    r1 = jnp.where((pos >= 0) & (pos < ml), r1, 0.0)
    z = (jnp.dot(_shift_dn(r1), k20_ref[...],
                 preferred_element_type=jnp.float32)
         + jnp.dot(r1, k21_ref[...], preferred_element_type=jnp.float32)
         + jnp.dot(_shift_up(r1), k22_ref[...],
                   preferred_element_type=jnp.float32)
         + c2b_ref[...])

    s0 = z[:, 0:1]
    s1 = z[:, 1:2]
    v0 = jnp.where((pos >= 0) & (pos < ln) & (pos <= off), s0, NEG_INF)
    v1 = jnp.where((pos < ln) & (pos >= off), s1, NEG_INF)

    @pl.when(h >= 2)
    def _():
        out_copy(a, slot).wait()

    win_s[slot] = jnp.concatenate([v0, v1], axis=1)
    cp2 = pltpu.make_async_copy(win_s.at[slot],
                                out_hbm.at[h, pl.ds(a, LP), :], sem2.at[slot])
    cp2.start()

    @pl.when(h == n - 1)
    def _():
        out_copy(a, slot).wait()
        out_copy(a, nslot).wait()


@functools.partial(jax.jit, static_argnames=())
def _impl(sent_id, words, heads_ids, emb, W1, b1, W2, b2, W3, b3,
          c1w, c1b, c2w, c2b):
    n_words = words.shape[0]
    n_heads = heads_ids.shape[0]
    input_size = words.shape[1]
    hidden = W1.shape[0]
    h2dim = W2.shape[0]
    demb = W3.shape[0]
    L = MAX_SENT_LEN

    heads_ids = heads_ids.astype(jnp.int32)
    sid_h = sent_id[heads_ids]
    starts = jnp.searchsorted(sent_id, sid_h, side='left').astype(jnp.int32)
    ends = jnp.searchsorted(sent_id, sid_h, side='right').astype(jnp.int32)
    lens = ends - starts
    offs = heads_ids - starts
    max_len = jnp.max(lens).reshape((1,))

    W1hT = W1[:, :input_size].T
    W1wT = W1[:, input_size:2 * input_size].T
    W1eT = W1[:, 2 * input_size:].T
    W2T = W2.T
    W3T = W3.T
    cc = c1w.shape[0]
    k1 = [jnp.pad(c1w[:, :, k].T, ((0, 0), (0, 128 - cc))) for k in range(3)]
    k2 = [jnp.pad(c2w[:, :, k].T, ((0, 128 - cc), (0, 126))) for k in range(3)]
    b1r = b1.reshape(1, hidden)
    b2r = b2.reshape(1, h2dim)
    b3r = b3.reshape(1, demb)
    c1br = jnp.pad(c1b, (0, 128 - cc)).reshape(1, 128)
    c2br = jnp.pad(c2b, (0, 126)).reshape(1, 128)

    npad = n_words + L
    nblk = n_words // L
    wordproj = pl.pallas_call(
        _wordproj_kernel,
        grid=(nblk + 1,),
        in_specs=[
            pl.BlockSpec((L, input_size),
                         lambda i: (jnp.minimum(i, nblk - 1), 0)),
            pl.BlockSpec((input_size, hidden), lambda i: (0, 0)),
        ],
        out_specs=pl.BlockSpec((L, hidden), lambda i: (i, 0)),
        out_shape=jax.ShapeDtypeStruct((npad, hidden), jnp.float32),
    )(words, W1wT)

    trev = pl.pallas_call(
        _trev_kernel,
        in_specs=[pl.BlockSpec(emb.shape, lambda: (0, 0)),
                  pl.BlockSpec((demb, hidden), lambda: (0, 0))],
        out_specs=pl.BlockSpec((8, TBL, hidden), lambda: (0, 0, 0)),
        out_shape=jax.ShapeDtypeStruct((8, TBL, hidden), jnp.float32),
    )(emb, W1eT)

    init = jnp.full((n_heads, npad, 2), NEG_INF, dtype=jnp.float32)

    grid_spec = pltpu.PrefetchScalarGridSpec(
        num_scalar_prefetch=5,
        grid=(n_heads,),
        in_specs=[
            pl.BlockSpec(memory_space=pl.ANY),
            pl.BlockSpec((8, TBL, hidden), lambda h, *_: (0, 0, 0)),
            pl.BlockSpec((1, 1, input_size),
                         lambda h, s, l, o, hid, ml: (hid[h], 0, 0)),
            pl.BlockSpec((input_size, hidden), lambda h, *_: (0, 0)),
            pl.BlockSpec((hidden, h2dim), lambda h, *_: (0, 0)),
            pl.BlockSpec((h2dim, demb), lambda h, *_: (0, 0)),
            pl.BlockSpec((demb, 128), lambda h, *_: (0, 0)),
            pl.BlockSpec((demb, 128), lambda h, *_: (0, 0)),
            pl.BlockSpec((demb, 128), lambda h, *_: (0, 0)),
            pl.BlockSpec((128, 128), lambda h, *_: (0, 0)),
            pl.BlockSpec((128, 128), lambda h, *_: (0, 0)),
            pl.BlockSpec((128, 128), lambda h, *_: (0, 0)),
            pl.BlockSpec((1, hidden), lambda h, *_: (0, 0)),
            pl.BlockSpec((1, h2dim), lambda h, *_: (0, 0)),
            pl.BlockSpec((1, demb), lambda h, *_: (0, 0)),
            pl.BlockSpec((1, 128), lambda h, *_: (0, 0)),
            pl.BlockSpec((1, 128), lambda h, *_: (0, 0)),
            pl.BlockSpec(memory_space=pl.ANY),
        ],
        out_specs=pl.BlockSpec(memory_space=pl.ANY),
        scratch_shapes=[
            pltpu.VMEM((2, L + 8, hidden), jnp.float32),
            pltpu.VMEM((2, L + 8, 2), jnp.float32),
            pltpu.SemaphoreType.DMA((2,)),
            pltpu.SemaphoreType.DMA((2,)),
        ],
    )

    out = pl.pallas_call(
        _head_kernel,
        grid_spec=grid_spec,
        out_shape=jax.ShapeDtypeStruct((n_heads, npad, 2), jnp.float32),
        input_output_aliases={22: 0},
        compiler_params=pltpu.CompilerParams(
            dimension_semantics=("arbitrary",)),
    )(starts, lens, offs, heads_ids, max_len,
      wordproj, trev, words.reshape(n_words, 1, input_size), W1hT, W2T, W3T,
      k1[0], k1[1], k1[2], k2[0], k2[1], k2[2],
      b1r, b2r, b3r, c1br, c2br, init)

    return out[:, :n_words, :]


def kernel(sent_id, words, heads_ids, emb, W1, b1, W2, b2, W3, b3,
           c1w, c1b, c2w, c2b):
    return _impl(sent_id, words, heads_ids, emb, W1, b1, W2, b2, W3, b3,
                 c1w, c1b, c2w, c2b)


# speedup vs baseline: 96.7097x; 1.4406x over previous
import functools

import jax
import jax.numpy as jnp
from jax import lax
from jax.experimental import pallas as pl
from jax.experimental.pallas import tpu as pltpu

MAX_DIST = 128
MAX_SENT_LEN = 256
NEG_INF = float('-inf')


def _wordproj_kernel(x_ref, w_ref, o_ref):
    o_ref[...] = jnp.dot(x_ref[...], w_ref[...],
                         preferred_element_type=jnp.float32)


TBL = 800
TXT = 808
TSH = 519


def _trev_kernel(emb_ref, w1e_ref, o_ref):
    ep = jnp.dot(emb_ref[...], w1e_ref[...],
                 preferred_element_type=jnp.float32)
    j = lax.broadcasted_iota(jnp.int32, (TXT, MAX_DIST), 0)
    i = lax.broadcasted_iota(jnp.int32, (TXT, MAX_DIST), 1)
    e = (TSH + (MAX_DIST - 2) // 2) - j
    tgt = jnp.where((e >= 0) & (e <= MAX_DIST - 2), e, MAX_DIST - 1)
    sel = (i == tgt).astype(jnp.float32)
    text = jnp.dot(sel, ep, preferred_element_type=jnp.float32)
    for v in range(8):
        o_ref[v, :, :] = text[v:v + TBL, :]


def _shift_dn(x):
    z = jnp.zeros((1, x.shape[1]), jnp.float32)
    return jnp.concatenate([z, x[:-1, :]], axis=0)


def _shift_up(x):
    z = jnp.zeros((1, x.shape[1]), jnp.float32)
    return jnp.concatenate([x[1:, :], z], axis=0)


def _head_kernel(starts_r, lens_r, offs_r, hids_r, ml_r,
                 wp_hbm, trev_ref, hw_ref, w1h_ref, w2_ref, w3_ref,
                 k10_ref, k11_ref, k12_ref, k20_ref, k21_ref, k22_ref,
                 b1_ref, b2_ref, b3_ref, c1b_ref, c2b_ref,
                 init_hbm, out_hbm,
                 wp_s, win_s, sem1, sem2):
    del init_hbm
    h = pl.program_id(0)
    n = pl.num_programs(0)
    start = starts_r[h]
    ln = lens_r[h]
    off = offs_r[h]
    ml = ml_r[0]

    L = MAX_SENT_LEN
    LP = L + 8
    nw = wp_hbm.shape[0]
    a = jnp.minimum((start // 8) * 8, nw - (L + 8))
    r = start - a
    slot = lax.rem(h, 2)
    nslot = 1 - slot

    def in_copy(idx, sl):
        aa = jnp.minimum((starts_r[idx] // 8) * 8, nw - (L + 8))
        return pltpu.make_async_copy(wp_hbm.at[pl.ds(aa, LP), :],
                                     wp_s.at[sl], sem1.at[sl])

    def out_copy(base_row, sl):
        return pltpu.make_async_copy(win_s.at[sl],
                                     out_hbm.at[0, pl.ds(base_row, LP), :],
                                     sem2.at[sl])

    @pl.when(h == 0)
    def _():
        in_copy(h, slot).start()

    @pl.when(h + 1 < n)
    def _():
        in_copy(h + 1, nslot).start()

    in_copy(h, slot).wait()

    base = jnp.clip(TSH - off - r, 0, TBL - LP)
    b_hi = (base // 8) * 8
    b_lo = base - b_hi
    e1 = trev_ref[b_lo, pl.ds(b_hi, LP), :]
    hp = jnp.dot(hw_ref[0], w1h_ref[...],
                 preferred_element_type=jnp.float32)
    pos = lax.broadcasted_iota(jnp.int32, (LP, 1), 0) - r

    b1 = b1_ref[...]
    pre = wp_s[slot] + e1 + hp + b1
    h1 = jnp.where((pos >= 0) & (pos < ln),
                   jnp.maximum(pre, 0.0), jnp.maximum(b1, 0.0))
    h2 = jnp.maximum(
        jnp.dot(h1, w2_ref[...], preferred_element_type=jnp.float32)
        + b2_ref[...], 0.0)
    h3 = jnp.dot(h2, w3_ref[...], preferred_element_type=jnp.float32) \
        + b3_ref[...]
    h3 = jnp.where((pos >= 0) & (pos < ml), h3, 0.0)

    r1 = (jnp.dot(_shift_dn(h3), k10_ref[...],
                  preferred_element_type=jnp.float32)
          + jnp.dot(h3, k11_ref[...], preferred_element_type=jnp.float32)
          + jnp.dot(_shift_up(h3), k12_ref[...],
                    preferred_element_type=jnp.float32)
          + c1b_ref[...])
    r1 = jnp.where((pos >= 0) & (pos < ml), r1, 0.0)
    z = (jnp.dot(_shift_dn(r1), k20_ref[...],
                 preferred_element_type=jnp.float32)
         + jnp.dot(r1, k21_ref[...], preferred_element_type=jnp.float32)
         + jnp.dot(_shift_up(r1), k22_ref[...],
                   preferred_element_type=jnp.float32)
         + c2b_ref[...])

    s0 = z[:, 0:1]
    s1 = z[:, 1:2]
    v0 = jnp.where((pos >= 0) & (pos < ln) & (pos <= off), s0, NEG_INF)
    v1 = jnp.where((pos < ln) & (pos >= off), s1, NEG_INF)

    @pl.when(h >= 2)
    def _():
        out_copy(a, slot).wait()

    win_s[slot] = jnp.concatenate([v0, v1], axis=1)
    cp2 = pltpu.make_async_copy(win_s.at[slot],
                                out_hbm.at[h, pl.ds(a, LP), :], sem2.at[slot])
    cp2.start()

    @pl.when(h == n - 1)
    def _():
        out_copy(a, slot).wait()
        out_copy(a, nslot).wait()


@functools.partial(jax.jit, static_argnames=())
def _impl(sent_id, words, heads_ids, emb, W1, b1, W2, b2, W3, b3,
          c1w, c1b, c2w, c2b):
    n_words = words.shape[0]
    n_heads = heads_ids.shape[0]
    input_size = words.shape[1]
    hidden = W1.shape[0]
    h2dim = W2.shape[0]
    demb = W3.shape[0]
    L = MAX_SENT_LEN

    heads_ids = heads_ids.astype(jnp.int32)
    sid_h = sent_id[heads_ids]
    starts = jnp.searchsorted(sent_id, sid_h, side='left').astype(jnp.int32)
    ends = jnp.searchsorted(sent_id, sid_h, side='right').astype(jnp.int32)
    lens = ends - starts
    offs = heads_ids - starts
    max_len = jnp.max(lens).reshape((1,))

    W1hT = W1[:, :input_size].T
    W1wT = W1[:, input_size:2 * input_size].T
    W1eT = W1[:, 2 * input_size:].T
    W2T = W2.T
    W3T = W3.T
    cc = c1w.shape[0]
    k1 = [jnp.pad(c1w[:, :, k].T, ((0, 0), (0, 128 - cc))) for k in range(3)]
    k2 = [jnp.pad(c2w[:, :, k].T, ((0, 128 - cc), (0, 126))) for k in range(3)]
    b1r = b1.reshape(1, hidden)
    b2r = b2.reshape(1, h2dim)
    b3r = b3.reshape(1, demb)
    c1br = jnp.pad(c1b, (0, 128 - cc)).reshape(1, 128)
    c2br = jnp.pad(c2b, (0, 126)).reshape(1, 128)

    nblk = n_words // L
    wordproj = pl.pallas_call(
        _wordproj_kernel,
        grid=(nblk,),
        in_specs=[
            pl.BlockSpec((L, input_size), lambda i: (i, 0)),
            pl.BlockSpec((input_size, hidden), lambda i: (0, 0)),
        ],
        out_specs=pl.BlockSpec((L, hidden), lambda i: (i, 0)),
        out_shape=jax.ShapeDtypeStruct((n_words, hidden), jnp.float32),
    )(words, W1wT)

    trev = pl.pallas_call(
        _trev_kernel,
        in_specs=[pl.BlockSpec(emb.shape, lambda: (0, 0)),
                  pl.BlockSpec((demb, hidden), lambda: (0, 0))],
        out_specs=pl.BlockSpec((8, TBL, hidden), lambda: (0, 0, 0)),
        out_shape=jax.ShapeDtypeStruct((8, TBL, hidden), jnp.float32),
    )(emb, W1eT)

    init = jnp.full((n_heads, n_words, 2), NEG_INF, dtype=jnp.float32)

    grid_spec = pltpu.PrefetchScalarGridSpec(
        num_scalar_prefetch=5,
        grid=(n_heads,),
        in_specs=[
            pl.BlockSpec(memory_space=pl.ANY),
            pl.BlockSpec((8, TBL, hidden), lambda h, *_: (0, 0, 0)),
            pl.BlockSpec((1, 1, input_size),
                         lambda h, s, l, o, hid, ml: (hid[h], 0, 0)),
            pl.BlockSpec((input_size, hidden), lambda h, *_: (0, 0)),
            pl.BlockSpec((hidden, h2dim), lambda h, *_: (0, 0)),
            pl.BlockSpec((h2dim, demb), lambda h, *_: (0, 0)),
            pl.BlockSpec((demb, 128), lambda h, *_: (0, 0)),
            pl.BlockSpec((demb, 128), lambda h, *_: (0, 0)),
            pl.BlockSpec((demb, 128), lambda h, *_: (0, 0)),
            pl.BlockSpec((128, 128), lambda h, *_: (0, 0)),
            pl.BlockSpec((128, 128), lambda h, *_: (0, 0)),
            pl.BlockSpec((128, 128), lambda h, *_: (0, 0)),
            pl.BlockSpec((1, hidden), lambda h, *_: (0, 0)),
            pl.BlockSpec((1, h2dim), lambda h, *_: (0, 0)),
            pl.BlockSpec((1, demb), lambda h, *_: (0, 0)),
            pl.BlockSpec((1, 128), lambda h, *_: (0, 0)),
            pl.BlockSpec((1, 128), lambda h, *_: (0, 0)),
            pl.BlockSpec(memory_space=pl.ANY),
        ],
        out_specs=pl.BlockSpec(memory_space=pl.ANY),
        scratch_shapes=[
            pltpu.VMEM((2, L + 8, hidden), jnp.float32),
            pltpu.VMEM((2, L + 8, 2), jnp.float32),
            pltpu.SemaphoreType.DMA((2,)),
            pltpu.SemaphoreType.DMA((2,)),
        ],
    )

    out = pl.pallas_call(
        _head_kernel,
        grid_spec=grid_spec,
        out_shape=jax.ShapeDtypeStruct((n_heads, n_words, 2), jnp.float32),
        input_output_aliases={22: 0},
        compiler_params=pltpu.CompilerParams(
            dimension_semantics=("arbitrary",)),
    )(starts, lens, offs, heads_ids, max_len,
      wordproj, trev, words.reshape(n_words, 1, input_size), W1hT, W2T, W3T,
      k1[0], k1[1], k1[2], k2[0], k2[1], k2[2],
      b1r, b2r, b3r, c1br, c2br, init)

    return out


def kernel(sent_id, words, heads_ids, emb, W1, b1, W2, b2, W3, b3,
           c1w, c1b, c2w, c2b):
    return _impl(sent_id, words, heads_ids, emb, W1, b1, W2, b2, W3, b3,
                 c1w, c1b, c2w, c2b)
